# baseline (speedup 1.0000x reference)
import functools
import jax
import jax.numpy as jnp
from jax.experimental import pallas as pl
from jax.experimental.pallas import tpu as pltpu


def _scatter_body(dst_ref, src_ref, table_ref, acc_ref, *, chunk, half,
                  base):
    garbage = acc_ref.shape[0] - 1

    @pl.when(pl.program_id(0) == 0)
    def _():
        acc_ref[...] = jnp.zeros_like(acc_ref)

    def body(e, _):
        d = dst_ref[0, 0, e] - base
        s = src_ref[0, 0, e]
        in_range = (d >= 0) & (d < half)
        d_eff = jnp.where(in_range, d, garbage)
        row = table_ref[pl.ds(s, 1), :]
        acc_ref[pl.ds(d_eff, 1), :] = acc_ref[pl.ds(d_eff, 1), :] + row
        return 0

    jax.lax.fori_loop(0, chunk, body, 0)


def _scatter_add(dst_flat, src_idx, table, n_rows_out, chunk=4000,
                 n_split=2):
    E = dst_flat.shape[0]
    d = table.shape[1]
    nblk = E // chunk
    assert nblk * chunk == E
    half = n_rows_out // n_split
    assert half * n_split == n_rows_out
    rows = ((half + 1 + 7) // 8) * 8
    dst3 = dst_flat.reshape(nblk, 1, chunk)
    src3 = src_idx.reshape(nblk, 1, chunk)
    parts = []
    for g in range(n_split):
        parts.append(pl.pallas_call(
            functools.partial(_scatter_body, chunk=chunk, half=half,
                              base=g * half),
            grid=(nblk,),
            in_specs=[
                pl.BlockSpec((1, 1, chunk), lambda n: (n, 0, 0),
                             memory_space=pltpu.SMEM),
                pl.BlockSpec((1, 1, chunk), lambda n: (n, 0, 0),
                             memory_space=pltpu.SMEM),
                pl.BlockSpec((table.shape[0], d), lambda n: (0, 0)),
            ],
            out_specs=pl.BlockSpec((rows, d), lambda n: (0, 0)),
            out_shape=jax.ShapeDtypeStruct((rows, d), jnp.float32),
        )(dst3, src3, table)[:half])
    return jnp.concatenate(parts, axis=0)


def _mm_stats_body(x_ref, w_ref, b_ref, out_ref, s_ref, sq_ref):
    x = x_ref[...]
    y = jnp.dot(x, w_ref[...], preferred_element_type=jnp.float32)
    y = y + b_ref[0:1, :]
    out_ref[...] = y
    rb = y.shape[0]
    part = jnp.sum(y.reshape(rb // 8, 8, y.shape[1]), axis=0)
    part_sq = jnp.sum((y * y).reshape(rb // 8, 8, y.shape[1]), axis=0)

    @pl.when(pl.program_id(0) == 0)
    def _():
        s_ref[...] = jnp.zeros_like(s_ref)
        sq_ref[...] = jnp.zeros_like(sq_ref)

    s_ref[...] += part
    sq_ref[...] += part_sq


def _mm_stats(x, w, b, rb=1000):
    n, d = x.shape
    dout = w.shape[1]
    b2 = jnp.broadcast_to(b[None, :], (8, dout))
    return pl.pallas_call(
        _mm_stats_body,
        grid=(n // rb,),
        in_specs=[
            pl.BlockSpec((rb, d), lambda r: (r, 0)),
            pl.BlockSpec((d, dout), lambda r: (0, 0)),
            pl.BlockSpec((8, dout), lambda r: (0, 0)),
        ],
        out_specs=[
            pl.BlockSpec((rb, dout), lambda r: (r, 0)),
            pl.BlockSpec((8, dout), lambda r: (0, 0)),
            pl.BlockSpec((8, dout), lambda r: (0, 0)),
        ],
        out_shape=[
            jax.ShapeDtypeStruct((n, dout), jnp.float32),
            jax.ShapeDtypeStruct((8, dout), jnp.float32),
            jax.ShapeDtypeStruct((8, dout), jnp.float32),
        ],
    )(x, w, b2)


def _conv_stats_body(agg_ref, w_ref, out_ref, s_ref, sq_ref):
    y = jnp.dot(agg_ref[...], w_ref[0], preferred_element_type=jnp.float32)
    out_ref[...] = y
    rb = y.shape[0]
    part = jnp.sum(y.reshape(rb // 8, 8, y.shape[1]), axis=0)
    part_sq = jnp.sum((y * y).reshape(rb // 8, 8, y.shape[1]), axis=0)

    @pl.when(pl.program_id(1) == 0)
    def _():
        s_ref[...] = jnp.zeros_like(s_ref)
        sq_ref[...] = jnp.zeros_like(sq_ref)

    s_ref[...] += part
    sq_ref[...] += part_sq


def _conv_stats(agg_flat, w_all, n, rb=1000):
    classes, d, dout = w_all.shape
    nr = n // rb
    out, s, sq = pl.pallas_call(
        _conv_stats_body,
        grid=(classes, nr),
        in_specs=[
            pl.BlockSpec((rb, d), lambda c, r: (c * nr + r, 0)),
            pl.BlockSpec((1, d, dout), lambda c, r: (c, 0, 0)),
        ],
        out_specs=[
            pl.BlockSpec((rb, dout), lambda c, r: (r, c)),
            pl.BlockSpec((8, dout), lambda c, r: (0, c)),
            pl.BlockSpec((8, dout), lambda c, r: (0, c)),
        ],
        out_shape=[
            jax.ShapeDtypeStruct((n, classes * dout), jnp.float32),
            jax.ShapeDtypeStruct((8, classes * dout), jnp.float32),
            jax.ShapeDtypeStruct((8, classes * dout), jnp.float32),
        ],
    )(agg_flat, w_all)
    return out, s, sq


def _final_body(fu_ref, hu_ref, fi_ref, hi_ref,
                sc_fu_ref, sh_fu_ref, sc_hu_ref, sh_hu_ref,
                sc_fi_ref, sh_fi_ref, sc_hi_ref, sh_hi_ref,
                wfu_ref, whu_ref, wfi_ref, whi_ref, kall_ref,
                out_ref, *, classes, de):
    fu = jnp.maximum(fu_ref[...] * sc_fu_ref[0:1, :] + sh_fu_ref[0:1, :], 0.)
    hu = jnp.maximum(hu_ref[...] * sc_hu_ref[0:1, :] + sh_hu_ref[0:1, :], 0.)
    fi = jnp.maximum(fi_ref[...] * sc_fi_ref[0:1, :] + sh_fi_ref[0:1, :], 0.)
    hi = jnp.maximum(hi_ref[...] * sc_hi_ref[0:1, :] + sh_hi_ref[0:1, :], 0.)
    ue = jnp.dot(fu, wfu_ref[...], preferred_element_type=jnp.float32)
    ue = ue + jnp.dot(hu, whu_ref[...], preferred_element_type=jnp.float32)
    ue = jnp.maximum(ue, 0.)
    ie = jnp.dot(fi, wfi_ref[...], preferred_element_type=jnp.float32)
    ie = ie + jnp.dot(hi, whi_ref[...], preferred_element_type=jnp.float32)
    ie = jnp.maximum(ie, 0.)
    u2 = jnp.dot(ue, kall_ref[...], preferred_element_type=jnp.float32)
    cols = []
    for c in range(classes):
        cols.append(jnp.sum(u2[:, c * de:(c + 1) * de] * ie,
                            axis=1, keepdims=True))
    out_ref[...] = jnp.concatenate(cols, axis=1)


def _scale_shift(g, b, s, sq, n, eps=1e-3):
    m = jnp.sum(s, axis=0) / n
    v = jnp.sum(sq, axis=0) / n - m * m
    scale = g / jnp.sqrt(v + eps)
    shift = b - m * scale
    dout = scale.shape[0]
    return (jnp.broadcast_to(scale[None, :], (8, dout)),
            jnp.broadcast_to(shift[None, :], (8, dout)))


def kernel(u_features, v_features, edge_user, edge_item, edge_rating,
           W_uf, b_uf, W_if, b_if, W_uc, W_ic,
           gamma_fu, beta_fu, gamma_hu, beta_hu,
           gamma_fi, beta_fi, gamma_hi, beta_hi,
           W_f_user, W_h_user, W_f_item, W_h_item,
           decoder, wc):
    n_users, d = u_features.shape
    n_items = v_features.shape[0]
    classes = W_uc.shape[0]
    de = W_f_item.shape[1]

    dst_u = edge_rating * n_users + edge_user
    dst_i = edge_rating * n_items + edge_item

    agg_u = _scatter_add(dst_u, edge_item, v_features, classes * n_users)
    agg_i = _scatter_add(dst_i, edge_user, u_features, classes * n_items)

    f_user, s_fu, sq_fu = _mm_stats(u_features, W_uf, b_uf)
    f_item, s_fi, sq_fi = _mm_stats(v_features, W_if, b_if)
    h_user, s_hu, sq_hu = _conv_stats(agg_u, W_uc, n_users)
    h_item, s_hi, sq_hi = _conv_stats(agg_i, W_ic, n_items)

    sc_fu, sh_fu = _scale_shift(gamma_fu, beta_fu, s_fu, sq_fu, n_users)
    sc_hu, sh_hu = _scale_shift(gamma_hu, beta_hu, s_hu, sq_hu, n_users)
    sc_fi, sh_fi = _scale_shift(gamma_fi, beta_fi, s_fi, sq_fi, n_items)
    sc_hi, sh_hi = _scale_shift(gamma_hi, beta_hi, s_hi, sq_hi, n_items)

    kall = jnp.concatenate(
        [jnp.einsum('k,kab->ab', wc[c], decoder) for c in range(classes)],
        axis=1)

    rb = 1000
    dh = classes * W_uc.shape[2]
    logits = pl.pallas_call(
        functools.partial(_final_body, classes=classes, de=de),
        grid=(n_users // rb,),
        in_specs=[
            pl.BlockSpec((rb, d), lambda r: (r, 0)),
            pl.BlockSpec((rb, dh), lambda r: (r, 0)),
            pl.BlockSpec((rb, d), lambda r: (r, 0)),
            pl.BlockSpec((rb, dh), lambda r: (r, 0)),
        ] + [pl.BlockSpec(x, lambda r: (0, 0)) for x in
             [(8, d), (8, d), (8, dh), (8, dh),
              (8, d), (8, d), (8, dh), (8, dh),
              (d, de), (dh, de), (d, de), (dh, de),
              (de, classes * de)]],
        out_specs=pl.BlockSpec((rb, classes), lambda r: (r, 0)),
        out_shape=jax.ShapeDtypeStruct((n_users, classes), jnp.float32),
    )(f_user, h_user, f_item, h_item,
      sc_fu, sh_fu, sc_hu, sh_hu, sc_fi, sh_fi, sc_hi, sh_hi,
      W_f_user, W_h_user, W_f_item, W_h_item, kall)
    return logits
